# Initial kernel scaffold; baseline (speedup 1.0000x reference)
import jax
import jax.numpy as jnp
from jax.experimental import pallas as pl


def kernel(inputs, table):
    raise NotImplementedError("write your pallas kernel here")


# speedup vs baseline: 1.1578x; 1.1578x over previous
import jax
import jax.numpy as jnp
from jax import lax
from jax.experimental import pallas as pl
from jax.experimental.pallas import tpu as pltpu
from jax.experimental.pallas import tpu_sc as plsc

_NUM_FIELDS = 26
_VOCAB_PER_FIELD = 100000
_EMBED_DIM = 128
_BATCH = 4096
_N = _BATCH * _NUM_FIELDS
_NC = 2
_NS = 16
_NW = _NC * _NS
_BPW = _N // _NW
_CH = 128
_NCH = _BPW // _CH


def _sc_body(inp_hbm, table_hbm, out_hbm, idx_v, rows_v, gsem):
    wid = lax.axis_index("s") * _NC + lax.axis_index("c")
    base = wid * _BPW

    pltpu.sync_copy(inp_hbm.at[wid], idx_v)

    def add_body(g, carry):
        j = g // 8
        k = g % 8
        pos = j * _CH + k * 16 + lax.iota(jnp.int32, 16)
        off = lax.rem(pos, _NUM_FIELDS) * _VOCAB_PER_FIELD
        idx_v[j, pl.ds(k * 16, 16)] = idx_v[j, pl.ds(k * 16, 16)] + off
        return carry

    lax.fori_loop(0, _NCH * 8, add_body, 0)

    def chunk_body(j, carry):
        pltpu.async_copy(table_hbm.at[idx_v.at[j]], rows_v, gsem).wait()
        pltpu.sync_copy(rows_v, out_hbm.at[pl.ds(base + j * _CH, _CH)])
        return carry

    lax.fori_loop(0, _NCH, chunk_body, 0)


def kernel(inputs, table):
    inp_r = inputs.reshape(_NW, _NCH, _CH)
    mesh = plsc.VectorSubcoreMesh(core_axis_name="c", subcore_axis_name="s")
    run = pl.kernel(
        _sc_body,
        out_type=jax.ShapeDtypeStruct((_N, _EMBED_DIM), jnp.float32),
        mesh=mesh,
        scratch_types=[
            pltpu.VMEM((_NCH, _CH), jnp.int32),
            pltpu.VMEM((_CH, _EMBED_DIM), jnp.float32),
            pltpu.SemaphoreType.DMA,
        ],
    )
    out = run(inp_r, table)
    return out.reshape(_BATCH, _NUM_FIELDS, _EMBED_DIM)


# speedup vs baseline: 1.2553x; 1.0843x over previous
import jax
import jax.numpy as jnp
from jax import lax
from jax.experimental import pallas as pl
from jax.experimental.pallas import tpu as pltpu
from jax.experimental.pallas import tpu_sc as plsc

_NUM_FIELDS = 26
_VOCAB_PER_FIELD = 100000
_EMBED_DIM = 128
_BATCH = 4096
_N = _BATCH * _NUM_FIELDS
_NC = 2
_NS = 16
_NW = _NC * _NS
_BPW = _N // _NW
_CH = 128
_NCH = _BPW // _CH


_NBUF = 2
_OUTER = _NCH // _NBUF


def _sc_body(inp_hbm, table_hbm, out_hbm, idx_v, rows_v, gsem0, gsem1,
             osem0, osem1):
    wid = lax.axis_index("s") * _NC + lax.axis_index("c")
    base = wid * _BPW
    gsems = (gsem0, gsem1)
    osems = (osem0, osem1)

    pltpu.sync_copy(inp_hbm.at[wid], idx_v)

    def add_body(g, carry):
        j = g // 8
        k = g % 8
        pos = j * _CH + k * 16 + lax.iota(jnp.int32, 16)
        off = lax.rem(pos, _NUM_FIELDS) * _VOCAB_PER_FIELD
        idx_v[j, pl.ds(k * 16, 16)] = idx_v[j, pl.ds(k * 16, 16)] + off
        return carry

    lax.fori_loop(0, _NCH * 8, add_body, 0)

    def gather_copy(j, b):
        return pltpu.make_async_copy(table_hbm.at[idx_v.at[j]], rows_v.at[b],
                                     gsems[b])

    def out_copy(j, b):
        return pltpu.make_async_copy(
            rows_v.at[b], out_hbm.at[pl.ds(base + j * _CH, _CH)], osems[b])

    for b in range(_NBUF):
        gather_copy(b, b).start()

    def outer(g, carry):
        j0 = g * _NBUF
        for b in range(_NBUF):
            gather_copy(j0 + b, b).wait()
            out_copy(j0 + b, b).start()
        for b in range(_NBUF):
            out_copy(j0 + b, b).wait()
            gather_copy(j0 + _NBUF + b, b).start()
        return carry

    lax.fori_loop(0, _OUTER - 1, outer, 0)

    j0 = (_OUTER - 1) * _NBUF
    for b in range(_NBUF):
        gather_copy(j0 + b, b).wait()
        out_copy(j0 + b, b).start()
    for b in range(_NBUF):
        out_copy(j0 + b, b).wait()


def kernel(inputs, table):
    inp_r = inputs.reshape(_NW, _NCH, _CH)
    mesh = plsc.VectorSubcoreMesh(core_axis_name="c", subcore_axis_name="s")
    run = pl.kernel(
        _sc_body,
        out_type=jax.ShapeDtypeStruct((_N, _EMBED_DIM), jnp.float32),
        mesh=mesh,
        scratch_types=[
            pltpu.VMEM((_NCH, _CH), jnp.int32),
            pltpu.VMEM((_NBUF, _CH, _EMBED_DIM), jnp.float32),
        ] + [pltpu.SemaphoreType.DMA] * (2 * _NBUF),
    )
    out = run(inp_r, table)
    return out.reshape(_BATCH, _NUM_FIELDS, _EMBED_DIM)


# speedup vs baseline: 2.0142x; 1.6045x over previous
import jax
import jax.numpy as jnp
from jax import lax
from jax.experimental import pallas as pl
from jax.experimental.pallas import tpu as pltpu
from jax.experimental.pallas import tpu_sc as plsc

_NUM_FIELDS = 26
_VOCAB_PER_FIELD = 100000
_EMBED_DIM = 128
_BATCH = 4096
_N = _BATCH * _NUM_FIELDS
_NC = 2
_NS = 16
_NW = _NC * _NS
_BPW = _N // _NW
_BATCH_PER_W = _BATCH // _NW
_CH = 104
_CHB = _CH // _NUM_FIELDS
_CHP = 128
_NCH = _BPW // _CH
_NBUF = 4
_OUTER = _NCH // _NBUF


def _sc_body(inp_hbm, table_hbm, out_hbm, idx_v, rows_v, gsem0, gsem1, gsem2,
             gsem3, osem0, osem1, osem2, osem3):
    wid = lax.axis_index("s") * _NC + lax.axis_index("c")
    gsems = (gsem0, gsem1, gsem2, gsem3)
    osems = (osem0, osem1, osem2, osem3)

    pltpu.sync_copy(inp_hbm.at[wid], idx_v)

    def add_body(j, carry):
        for k in range(_CHP // 16):
            off = lax.rem(k * 16 + lax.iota(jnp.int32, 16),
                          _NUM_FIELDS) * _VOCAB_PER_FIELD
            idx_v[j, pl.ds(k * 16, 16)] = idx_v[j, pl.ds(k * 16, 16)] + off
        return carry

    lax.fori_loop(0, _NCH, add_body, 0)

    def gather_copy(j, b):
        return pltpu.make_async_copy(
            table_hbm.at[idx_v.at[j, pl.ds(0, _CH)]], rows_v.at[b], gsems[b])

    def out_copy(j, b, t):
        bat = wid * _BATCH_PER_W + j * _CHB + t
        return pltpu.make_async_copy(
            rows_v.at[b, pl.ds(t * _NUM_FIELDS, _NUM_FIELDS)],
            out_hbm.at[bat], osems[b])

    for b in range(_NBUF):
        gather_copy(b, b).start()

    def outer(g, carry):
        j0 = g * _NBUF
        for b in range(_NBUF):
            gather_copy(j0 + b, b).wait()
            for t in range(_CHB):
                out_copy(j0 + b, b, t).start()
        for b in range(_NBUF):
            for t in range(_CHB):
                out_copy(j0 + b, b, t).wait()
            gather_copy(j0 + _NBUF + b, b).start()
        return carry

    lax.fori_loop(0, _OUTER - 1, outer, 0)

    j0 = (_OUTER - 1) * _NBUF
    for b in range(_NBUF):
        gather_copy(j0 + b, b).wait()
        for t in range(_CHB):
            out_copy(j0 + b, b, t).start()
    for b in range(_NBUF):
        for t in range(_CHB):
            out_copy(j0 + b, b, t).wait()


def kernel(inputs, table):
    inp_p = jnp.pad(inputs.reshape(_NW * _NCH, _CH),
                    ((0, 0), (0, _CHP - _CH))).reshape(_NW, _NCH, _CHP)
    mesh = plsc.VectorSubcoreMesh(core_axis_name="c", subcore_axis_name="s")
    run = pl.kernel(
        _sc_body,
        out_type=jax.ShapeDtypeStruct((_BATCH, _NUM_FIELDS, _EMBED_DIM),
                                      jnp.float32),
        mesh=mesh,
        scratch_types=[
            pltpu.VMEM((_NCH, _CHP), jnp.int32),
            pltpu.VMEM((_NBUF, _CH, _EMBED_DIM), jnp.float32),
        ] + [pltpu.SemaphoreType.DMA] * (2 * _NBUF),
    )
    return run(inp_p, table)


# speedup vs baseline: 3.4446x; 1.7101x over previous
import jax
import jax.numpy as jnp
from jax import lax
from jax.experimental import pallas as pl
from jax.experimental.pallas import tpu as pltpu
from jax.experimental.pallas import tpu_sc as plsc

_NUM_FIELDS = 26
_VOCAB_PER_FIELD = 100000
_EMBED_DIM = 128
_BATCH = 4096
_NC = 2
_NS = 16
_NW = _NC * _NS
_BPW = _BATCH // _NW
_NBUF = 2
_OUTER = _NUM_FIELDS // _NBUF


def _sc_body(inp_hbm, table_hbm, out_hbm, idx_v, rows_v, gsem0, gsem1,
             osem0, osem1):
    wid = lax.axis_index("s") * _NC + lax.axis_index("c")
    b0 = wid * _BPW
    gsems = (gsem0, gsem1)
    osems = (osem0, osem1)

    pltpu.sync_copy(inp_hbm.at[:, pl.ds(b0, _BPW)], idx_v)

    def add_body(f, carry):
        off = f * _VOCAB_PER_FIELD
        for k in range(_BPW // 16):
            idx_v[f, pl.ds(k * 16, 16)] = idx_v[f, pl.ds(k * 16, 16)] + off
        return carry

    lax.fori_loop(0, _NUM_FIELDS, add_body, 0)

    def gather_copy(f, b):
        return pltpu.make_async_copy(
            table_hbm.at[idx_v.at[f]], rows_v.at[b], gsems[b])

    def out_copy(f, b):
        return pltpu.make_async_copy(
            rows_v.at[b], out_hbm.at[f, pl.ds(b0, _BPW)], osems[b])

    for b in range(_NBUF):
        gather_copy(b, b).start()

    def outer(g, carry):
        f0 = g * _NBUF
        for b in range(_NBUF):
            gather_copy(f0 + b, b).wait()
            out_copy(f0 + b, b).start()
        for b in range(_NBUF):
            out_copy(f0 + b, b).wait()
            gather_copy(f0 + _NBUF + b, b).start()
        return carry

    lax.fori_loop(0, _OUTER - 1, outer, 0)

    f0 = (_OUTER - 1) * _NBUF
    for b in range(_NBUF):
        gather_copy(f0 + b, b).wait()
        out_copy(f0 + b, b).start()
    for b in range(_NBUF):
        out_copy(f0 + b, b).wait()


def kernel(inputs, table):
    inp_t = inputs.T
    mesh = plsc.VectorSubcoreMesh(core_axis_name="c", subcore_axis_name="s")
    run = pl.kernel(
        _sc_body,
        out_type=jax.ShapeDtypeStruct((_NUM_FIELDS, _BATCH, _EMBED_DIM),
                                      jnp.float32),
        mesh=mesh,
        scratch_types=[
            pltpu.VMEM((_NUM_FIELDS, _BPW), jnp.int32),
            pltpu.VMEM((_NBUF, _BPW, _EMBED_DIM), jnp.float32),
        ] + [pltpu.SemaphoreType.DMA] * (2 * _NBUF),
    )
    out = run(inp_t, table)
    return out.transpose(1, 0, 2)


# speedup vs baseline: 3.6260x; 1.0527x over previous
import jax
import jax.numpy as jnp
from jax import lax
from jax.experimental import pallas as pl
from jax.experimental.pallas import tpu as pltpu
from jax.experimental.pallas import tpu_sc as plsc

_NUM_FIELDS = 26
_VOCAB_PER_FIELD = 100000
_EMBED_DIM = 128
_BATCH = 4096
_NC = 2
_NS = 16
_NW = _NC * _NS
_BPW = _BATCH // _NW
_CH = 64
_NCHUNK = _NUM_FIELDS * (_BPW // _CH)
_NBUF = 4
_OUTER = _NCHUNK // _NBUF


def _sc_body(inp_hbm, table_hbm, out_hbm, idx_v, rows_v, gsem0, gsem1, gsem2,
             gsem3, osem0, osem1, osem2, osem3):
    wid = lax.axis_index("s") * _NC + lax.axis_index("c")
    b0 = wid * _BPW
    gsems = (gsem0, gsem1, gsem2, gsem3)
    osems = (osem0, osem1, osem2, osem3)

    pltpu.sync_copy(inp_hbm.at[:, pl.ds(b0, _BPW)], idx_v)

    def add_body(f, carry):
        off = f * _VOCAB_PER_FIELD
        for k in range(_BPW // 16):
            idx_v[f, pl.ds(k * 16, 16)] = idx_v[f, pl.ds(k * 16, 16)] + off
        return carry

    lax.fori_loop(0, _NUM_FIELDS, add_body, 0)

    def gather_copy(j, b):
        f = j // 2
        h = lax.rem(j, 2) * _CH
        return pltpu.make_async_copy(
            table_hbm.at[idx_v.at[f, pl.ds(h, _CH)]], rows_v.at[b], gsems[b])

    def out_copy(j, b):
        f = j // 2
        h = lax.rem(j, 2) * _CH
        return pltpu.make_async_copy(
            rows_v.at[b], out_hbm.at[f, pl.ds(b0 + h, _CH)], osems[b])

    for b in range(_NBUF):
        gather_copy(b, b).start()

    def outer(g, carry):
        j0 = g * _NBUF
        for b in range(_NBUF):
            gather_copy(j0 + b, b).wait()
            out_copy(j0 + b, b).start()
        for b in range(_NBUF):
            out_copy(j0 + b, b).wait()
            gather_copy(j0 + _NBUF + b, b).start()
        return carry

    lax.fori_loop(0, _OUTER - 1, outer, 0)

    j0 = (_OUTER - 1) * _NBUF
    for b in range(_NBUF):
        gather_copy(j0 + b, b).wait()
        out_copy(j0 + b, b).start()
    for b in range(_NBUF):
        out_copy(j0 + b, b).wait()


def kernel(inputs, table):
    inp_t = inputs.T
    mesh = plsc.VectorSubcoreMesh(core_axis_name="c", subcore_axis_name="s")
    run = pl.kernel(
        _sc_body,
        out_type=jax.ShapeDtypeStruct((_NUM_FIELDS, _BATCH, _EMBED_DIM),
                                      jnp.float32),
        mesh=mesh,
        scratch_types=[
            pltpu.VMEM((_NUM_FIELDS, _BPW), jnp.int32),
            pltpu.VMEM((_NBUF, _CH, _EMBED_DIM), jnp.float32),
        ] + [pltpu.SemaphoreType.DMA] * (2 * _NBUF),
    )
    out = run(inp_t, table)
    return out.transpose(1, 0, 2)


# speedup vs baseline: 3.7263x; 1.0277x over previous
import jax
import jax.numpy as jnp
from jax import lax
from jax.experimental import pallas as pl
from jax.experimental.pallas import tpu as pltpu
from jax.experimental.pallas import tpu_sc as plsc

_NUM_FIELDS = 26
_VOCAB_PER_FIELD = 100000
_EMBED_DIM = 128
_BATCH = 4096
_NC = 2
_NS = 16
_NW = _NC * _NS
_BPW = _BATCH // _NW
_CH = 64
_NCHUNK = _NUM_FIELDS * (_BPW // _CH)
_NBUF = 13
_OUTER = _NCHUNK // _NBUF


def _sc_body(inp_hbm, table_hbm, out_hbm, idx_v, rows_v, *sems):
    wid = lax.axis_index("s") * _NC + lax.axis_index("c")
    b0 = wid * _BPW
    gsems = sems[:_NBUF]
    osems = sems[_NBUF:]

    pltpu.sync_copy(inp_hbm.at[:, pl.ds(b0, _BPW)], idx_v)

    def add_body(f, carry):
        off = f * _VOCAB_PER_FIELD
        for k in range(_BPW // 16):
            idx_v[f, pl.ds(k * 16, 16)] = idx_v[f, pl.ds(k * 16, 16)] + off
        return carry

    lax.fori_loop(0, _NUM_FIELDS, add_body, 0)

    def gather_copy(j, b):
        f = j // 2
        h = lax.rem(j, 2) * _CH
        return pltpu.make_async_copy(
            table_hbm.at[idx_v.at[f, pl.ds(h, _CH)]], rows_v.at[b], gsems[b])

    def out_copy(j, b):
        f = j // 2
        h = lax.rem(j, 2) * _CH
        return pltpu.make_async_copy(
            rows_v.at[b], out_hbm.at[f, pl.ds(b0 + h, _CH)], osems[b])

    for b in range(_NBUF):
        gather_copy(b, b).start()

    def outer(g, carry):
        j0 = g * _NBUF
        for b in range(_NBUF):
            gather_copy(j0 + b, b).wait()
            out_copy(j0 + b, b).start()
        for b in range(_NBUF):
            out_copy(j0 + b, b).wait()
            gather_copy(j0 + _NBUF + b, b).start()
        return carry

    lax.fori_loop(0, _OUTER - 1, outer, 0)

    j0 = (_OUTER - 1) * _NBUF
    for b in range(_NBUF):
        gather_copy(j0 + b, b).wait()
        out_copy(j0 + b, b).start()
    for b in range(_NBUF):
        out_copy(j0 + b, b).wait()


def kernel(inputs, table):
    inp_t = inputs.T
    mesh = plsc.VectorSubcoreMesh(core_axis_name="c", subcore_axis_name="s")
    run = pl.kernel(
        _sc_body,
        out_type=jax.ShapeDtypeStruct((_NUM_FIELDS, _BATCH, _EMBED_DIM),
                                      jnp.float32),
        mesh=mesh,
        scratch_types=[
            pltpu.VMEM((_NUM_FIELDS, _BPW), jnp.int32),
            pltpu.VMEM((_NBUF, _CH, _EMBED_DIM), jnp.float32),
        ] + [pltpu.SemaphoreType.DMA] * (2 * _NBUF),
    )
    out = run(inp_t, table)
    return out.transpose(1, 0, 2)


# speedup vs baseline: 3.8093x; 1.0223x over previous
import jax
import jax.numpy as jnp
from jax import lax
from jax.experimental import pallas as pl
from jax.experimental.pallas import tpu as pltpu
from jax.experimental.pallas import tpu_sc as plsc

_NUM_FIELDS = 26
_VOCAB_PER_FIELD = 100000
_EMBED_DIM = 128
_BATCH = 4096
_NC = 2
_NS = 16
_NW = _NC * _NS
_BPW = _BATCH // _NW
_NBUF = 7


def _fuse_row(idx_v, f):
    off = f * _VOCAB_PER_FIELD
    for k in range(_BPW // 16):
        idx_v[f, pl.ds(k * 16, 16)] = idx_v[f, pl.ds(k * 16, 16)] + off


def _sc_body(inp_hbm, table_hbm, out_hbm, idx_v, rows_v, *sems):
    wid = lax.axis_index("s") * _NC + lax.axis_index("c")
    b0 = wid * _BPW
    gsems = sems[:_NBUF]
    osems = sems[_NBUF:]

    pltpu.sync_copy(inp_hbm.at[:, pl.ds(b0, _BPW)], idx_v)

    def gather_copy(f, b):
        return pltpu.make_async_copy(
            table_hbm.at[idx_v.at[f]], rows_v.at[b], gsems[b])

    def out_copy(f, b):
        return pltpu.make_async_copy(
            rows_v.at[b], out_hbm.at[f, pl.ds(b0, _BPW)], osems[b])

    for b in range(_NBUF):
        _fuse_row(idx_v, b)
        gather_copy(b, b).start()
    for f in range(_NBUF, _NUM_FIELDS):
        _fuse_row(idx_v, f)

    for f in range(_NUM_FIELDS):
        b = f % _NBUF
        gather_copy(f, b).wait()
        out_copy(f, b).start()
        fn = f + _NBUF
        if fn < _NUM_FIELDS:
            out_copy(f, b).wait()
            gather_copy(fn, b).start()
    for f in range(_NUM_FIELDS - _NBUF, _NUM_FIELDS):
        out_copy(f, f % _NBUF).wait()


def kernel(inputs, table):
    inp_t = inputs.T
    mesh = plsc.VectorSubcoreMesh(core_axis_name="c", subcore_axis_name="s")
    run = pl.kernel(
        _sc_body,
        out_type=jax.ShapeDtypeStruct((_NUM_FIELDS, _BATCH, _EMBED_DIM),
                                      jnp.float32),
        mesh=mesh,
        scratch_types=[
            pltpu.VMEM((_NUM_FIELDS, _BPW), jnp.int32),
            pltpu.VMEM((_NBUF, _BPW, _EMBED_DIM), jnp.float32),
        ] + [pltpu.SemaphoreType.DMA] * (2 * _NBUF),
    )
    out = run(inp_t, table)
    return out.transpose(1, 0, 2)
